# baseline (speedup 1.0000x reference)
import functools

import jax
import jax.numpy as jnp
from jax import lax
from jax.experimental import pallas as pl
from jax.experimental.pallas import tpu as pltpu
from jax.experimental.pallas import tpu_sc as plsc

_NC = 2
_NSUB = 16
_NW = _NC * _NSUB


def _meta_body(xf_ref, rW_ref, rb_ref, tri_ref, sW1_ref, sW2_ref,
               ys_ref, pos0_ref, pos1_ref, g0x_ref, g1x_ref, be_ref, loss_ref,
               cnts_ref, lcnt_ref, off_ref, run_ref,
               *, T, E, NT, N, K, TB, NB):
    t = pl.program_id(0)

    xf = xf_ref[...]
    logits = jnp.dot(xf, rW_ref[...],
                     preferred_element_type=jnp.float32) + rb_ref[...]
    lane = lax.broadcasted_iota(jnp.int32, (T, E), 1)
    m0 = jnp.max(logits, axis=1, keepdims=True)
    i0 = jnp.argmax(logits, axis=1).reshape(T, 1)
    masked = jnp.where(lane == i0, -jnp.inf, logits)
    m1 = jnp.max(masked, axis=1, keepdims=True)
    i1 = jnp.argmax(masked, axis=1).reshape(T, 1)
    oh0 = (lane == i0).astype(jnp.float32)
    oh1 = (lane == i1).astype(jnp.float32)
    a_mat = oh0 + oh1
    colsum = jnp.sum(a_mat, axis=0, keepdims=True)

    c0 = jnp.sum((i0 == 0).astype(jnp.float32))
    c1 = jnp.sum((i1 == 1).astype(jnp.float32))
    lane_e = lax.broadcasted_iota(jnp.int32, (1, E), 1)
    lvec = jnp.where(lane_e == 0, c0, jnp.where(lane_e == 1, c1, 0.0))

    @pl.when(t == 0)
    def _():
        cnts_ref[...] = colsum
        lcnt_ref[...] = lvec

    @pl.when(jnp.logical_and(t > 0, t < NT))
    def _():
        cnts_ref[...] += colsum
        lcnt_ref[...] += lvec

    @pl.when(t == NT - 1)
    def _():
        ci = cnts_ref[...].astype(jnp.int32)
        cap = ((ci + TB - 1) // TB) * TB
        capf = cap.astype(jnp.float32)
        r8 = lax.broadcasted_iota(jnp.int32, (E, E), 0)
        c8 = lax.broadcasted_iota(jnp.int32, (E, E), 1)
        tri8 = (r8 < c8).astype(jnp.float32)
        off = jnp.dot(capf, tri8, preferred_element_type=jnp.float32)
        off_ref[...] = off
        bvec = lax.broadcasted_iota(jnp.int32, (64, 1), 0) * TB
        ge = (bvec >= off.astype(jnp.int32)).astype(jnp.int32)
        be = jnp.clip(jnp.sum(ge, axis=1, keepdims=True) - 1, 0, E - 1)
        be_ref[...] = be
        ec = lcnt_ref[...] / (N * K) + 1e-08
        loss_ref[...] = (-jnp.sum(ec * jnp.log(ec))).reshape(1, 1)

    @pl.when(t >= NT)
    def _phase_b():
        @pl.when(t == NT)
        def _():
            run_ref[...] = jnp.zeros_like(run_ref)

        d = jnp.exp(m1 - m0)
        g0 = 1.0 / (1.0 + d)
        g1 = d / (1.0 + d)
        rank = jnp.dot(tri_ref[...], a_mat, preferred_element_type=jnp.float32)
        basec = rank + run_ref[...] + off_ref[...]
        pos0_ref[...] = jnp.sum(basec * oh0, axis=1,
                                keepdims=True).astype(jnp.int32)
        pos1_ref[...] = jnp.sum(basec * oh1, axis=1,
                                keepdims=True).astype(jnp.int32)
        run_ref[...] += colsum
        g0x_ref[...] = jnp.broadcast_to(g0, g0x_ref.shape)
        g1x_ref[...] = jnp.broadcast_to(g1, g1x_ref.shape)

        xb = xf.astype(jnp.bfloat16)
        ys = None
        for n in range(sW1_ref.shape[0]):
            h = jnp.dot(xb, sW1_ref[n], preferred_element_type=jnp.float32)
            h = 0.5 * h * (1.0 + lax.erf(h * 0.7071067811865476))
            y = jnp.dot(h.astype(jnp.bfloat16), sW2_ref[n],
                        preferred_element_type=jnp.float32)
            ys = y if ys is None else ys + y
        ys_ref[...] = ys


def _ffn_body(be_sref, buf_ref, W1_ref, W2_ref, gb_ref, y_ref, *, TB):
    b = pl.program_id(0)
    e = be_sref[b]
    xb = buf_ref[...].astype(jnp.bfloat16)
    w1 = W1_ref[pl.ds(e, 1)][0]
    h = jnp.dot(xb, w1, preferred_element_type=jnp.float32)
    h = 0.5 * h * (1.0 + lax.erf(h * 0.7071067811865476))
    w2 = W2_ref[pl.ds(e, 1)][0]
    y = jnp.dot(h.astype(jnp.bfloat16), w2, preferred_element_type=jnp.float32)
    y_ref[...] = y * gb_ref[:, 0:1]


def kernel(x, router_W, router_b, eW1, eb1, eW2, eb2, sW1, sb1, sW2, sb2):
    B, S, DIM = x.shape
    E, _, F = eW1.shape
    NS = sW1.shape[0]
    K = 2
    N = B * S
    T = 512
    NT = N // T
    TB = 256
    CAP = N * K + E * TB
    NB = CAP // TB
    CHUNK = N // _NW

    xf = x.reshape(N, DIM)
    rb = router_b.reshape(1, E)
    tri = (jnp.arange(T)[:, None] > jnp.arange(T)[None, :]
           ).astype(jnp.float32)
    sW1b = sW1.astype(jnp.bfloat16)
    sW2b = sW2.astype(jnp.bfloat16)
    W1e = eW1.astype(jnp.bfloat16)
    W2e = eW2.astype(jnp.bfloat16)

    meta = functools.partial(_meta_body, T=T, E=E, NT=NT, N=N, K=K, TB=TB,
                             NB=NB)

    def _blk(t):
        return (lax.select(t < NT, 0, t - NT), 0)

    ys, pos0, pos1, g0x, g1x, be, loss = pl.pallas_call(
        meta,
        grid=(2 * NT,),
        in_specs=[
            pl.BlockSpec((T, DIM), lambda t: (lax.rem(t, NT), 0)),
            pl.BlockSpec((DIM, E), lambda t: (0, 0)),
            pl.BlockSpec((1, E), lambda t: (0, 0)),
            pl.BlockSpec((T, T), lambda t: (0, 0)),
            pl.BlockSpec((NS, DIM, F), lambda t: (0, 0, 0)),
            pl.BlockSpec((NS, F, DIM), lambda t: (0, 0, 0)),
        ],
        out_specs=[
            pl.BlockSpec((T, DIM), _blk),
            pl.BlockSpec((T, 1), _blk),
            pl.BlockSpec((T, 1), _blk),
            pl.BlockSpec((T, 128), _blk),
            pl.BlockSpec((T, 128), _blk),
            pl.BlockSpec((64, 1), lambda t: (0, 0)),
            pl.BlockSpec((1, 1), lambda t: (0, 0)),
        ],
        out_shape=[
            jax.ShapeDtypeStruct((N, DIM), jnp.float32),
            jax.ShapeDtypeStruct((N, 1), jnp.int32),
            jax.ShapeDtypeStruct((N, 1), jnp.int32),
            jax.ShapeDtypeStruct((N, 128), jnp.float32),
            jax.ShapeDtypeStruct((N, 128), jnp.float32),
            jax.ShapeDtypeStruct((64, 1), jnp.int32),
            jax.ShapeDtypeStruct((1, 1), jnp.float32),
        ],
        scratch_shapes=[
            pltpu.VMEM((1, E), jnp.float32),
            pltpu.VMEM((1, E), jnp.float32),
            pltpu.VMEM((1, E), jnp.float32),
            pltpu.VMEM((1, E), jnp.float32),
        ],
    )(xf, router_W, rb, tri, sW1b, sW2b)

    p0f = pos0.reshape(N)
    p1f = pos1.reshape(N)
    pos_sc = jnp.concatenate(
        [p0f.reshape(_NW, 2, CHUNK // 2), p1f.reshape(_NW, 2, CHUNK // 2)],
        axis=1)

    mesh = plsc.VectorSubcoreMesh(core_axis_name="c", subcore_axis_name="s")

    @functools.partial(
        pl.kernel, mesh=mesh,
        out_type=[
            jax.ShapeDtypeStruct((CAP, DIM), jnp.float32),
            jax.ShapeDtypeStruct((CAP, 128), jnp.float32),
        ],
        scratch_types=[
            pltpu.VMEM((CHUNK // 2, DIM), jnp.float32),
            pltpu.VMEM((CHUNK // 2, 128), jnp.float32),
            pltpu.VMEM((4, CHUNK // 2), jnp.int32),
            pltpu.SemaphoreType.DMA,
        ],
    )
    def _dispatch(x_hbm, psc_hbm, g0_hbm, g1_hbm, buf_hbm, gb_hbm,
                  rows_v, grow_v, idx_v, sem):
        wid = lax.axis_index("s") * _NC + lax.axis_index("c")
        base = wid * CHUNK
        pltpu.sync_copy(psc_hbm.at[wid], idx_v)
        half = CHUNK // 2
        for k in range(2):
            g_hbm = g0_hbm if k == 0 else g1_hbm
            for h in range(2):
                row = 2 * k + h
                pltpu.sync_copy(x_hbm.at[pl.ds(base + h * half, half)], rows_v)
                pltpu.async_copy(rows_v, buf_hbm.at[idx_v.at[row]], sem).wait()
                pltpu.sync_copy(g_hbm.at[pl.ds(base + h * half, half)], grow_v)
                pltpu.async_copy(grow_v, gb_hbm.at[idx_v.at[row]], sem).wait()

    bufX, gbuf = _dispatch(xf, pos_sc, g0x, g1x)

    ffn = functools.partial(_ffn_body, TB=TB)
    ybuf = pl.pallas_call(
        ffn,
        grid_spec=pltpu.PrefetchScalarGridSpec(
            num_scalar_prefetch=1,
            grid=(NB,),
            in_specs=[
                pl.BlockSpec((TB, DIM), lambda b, be_s: (b, 0)),
                pl.BlockSpec((E, DIM, F), lambda b, be_s: (0, 0, 0)),
                pl.BlockSpec((E, F, DIM), lambda b, be_s: (0, 0, 0)),
                pl.BlockSpec((TB, 128), lambda b, be_s: (b, 0)),
            ],
            out_specs=pl.BlockSpec((TB, DIM), lambda b, be_s: (b, 0)),
        ),
        out_shape=jax.ShapeDtypeStruct((CAP, DIM), jnp.float32),
    )(be.reshape(64), bufX, W1e, W2e, gbuf)

    @functools.partial(
        pl.kernel, mesh=mesh,
        out_type=jax.ShapeDtypeStruct((N, DIM), jnp.float32),
        scratch_types=[
            pltpu.VMEM((CHUNK,), jnp.int32),
            pltpu.VMEM((CHUNK,), jnp.int32),
            pltpu.VMEM((16, DIM), jnp.float32),
            pltpu.VMEM((16, DIM), jnp.float32),
            pltpu.VMEM((16, DIM), jnp.float32),
            pltpu.SemaphoreType.DMA,
            pltpu.SemaphoreType.DMA,
        ],
    )
    def _combine(ybuf_hbm, ys_hbm, p0_hbm, p1_hbm, out_hbm,
                 p0v, p1v, r0, r1, ov, sem0, sem1):
        wid = lax.axis_index("s") * _NC + lax.axis_index("c")
        base = wid * CHUNK
        pltpu.sync_copy(p0_hbm.at[pl.ds(base, CHUNK)], p0v)
        pltpu.sync_copy(p1_hbm.at[pl.ds(base, CHUNK)], p1v)
        for c in range(CHUNK // 16):
            cp0 = pltpu.async_copy(
                ybuf_hbm.at[p0v.at[pl.ds(c * 16, 16)]], r0, sem0)
            cp1 = pltpu.async_copy(
                ybuf_hbm.at[p1v.at[pl.ds(c * 16, 16)]], r1, sem1)
            pltpu.sync_copy(ys_hbm.at[pl.ds(base + c * 16, 16)], ov)
            cp0.wait()
            cp1.wait()

            def _add(j, _):
                row = j // (DIM // 16)
                col = lax.rem(j, DIM // 16) * 16
                ov[row, pl.ds(col, 16)] = (ov[row, pl.ds(col, 16)]
                                           + r0[row, pl.ds(col, 16)]
                                           + r1[row, pl.ds(col, 16)])
                return 0

            lax.fori_loop(0, 16 * (DIM // 16), _add, 0, unroll=8)
            pltpu.sync_copy(ov, out_hbm.at[pl.ds(base + c * 16, 16)])

    out = _combine(ybuf, ys, p0f, p1f)
    return out.reshape(B, S, DIM), loss[0, 0]


# speedup vs baseline: 1.0418x; 1.0418x over previous
import functools

import jax
import jax.numpy as jnp
from jax import lax
from jax.experimental import pallas as pl
from jax.experimental.pallas import tpu as pltpu
from jax.experimental.pallas import tpu_sc as plsc

_NC = 2
_NSUB = 16
_NW = _NC * _NSUB


def _meta_body(xf_ref, rW_ref, rb_ref, tri_ref, sW1_ref, sW2_ref,
               ys_ref, pos0_ref, pos1_ref, g0x_ref, g1x_ref, be_ref,
               loss_ref,
               cnts_ref, lcnt_ref, off_ref, run_ref,
               *, T, E, NT, N, K, TB, NB):
    t = pl.program_id(0)

    xf = xf_ref[...]
    logits = jnp.dot(xf, rW_ref[...],
                     preferred_element_type=jnp.float32) + rb_ref[...]
    lane = lax.broadcasted_iota(jnp.int32, (T, E), 1)
    m0 = jnp.max(logits, axis=1, keepdims=True)
    i0 = jnp.argmax(logits, axis=1).reshape(T, 1)
    masked = jnp.where(lane == i0, -jnp.inf, logits)
    m1 = jnp.max(masked, axis=1, keepdims=True)
    i1 = jnp.argmax(masked, axis=1).reshape(T, 1)
    oh0 = (lane == i0).astype(jnp.float32)
    oh1 = (lane == i1).astype(jnp.float32)
    a_mat = oh0 + oh1
    colsum = jnp.sum(a_mat, axis=0, keepdims=True)

    c0 = jnp.sum((i0 == 0).astype(jnp.float32))
    c1 = jnp.sum((i1 == 1).astype(jnp.float32))
    lane_e = lax.broadcasted_iota(jnp.int32, (1, E), 1)
    lvec = jnp.where(lane_e == 0, c0, jnp.where(lane_e == 1, c1, 0.0))

    @pl.when(t == 0)
    def _():
        cnts_ref[...] = colsum
        lcnt_ref[...] = lvec

    @pl.when(jnp.logical_and(t > 0, t < NT))
    def _():
        cnts_ref[...] += colsum
        lcnt_ref[...] += lvec

    @pl.when(t == NT - 1)
    def _():
        ci = cnts_ref[...].astype(jnp.int32)
        cap = ((ci + TB - 1) // TB) * TB
        capf = cap.astype(jnp.float32)
        r8 = lax.broadcasted_iota(jnp.int32, (E, E), 0)
        c8 = lax.broadcasted_iota(jnp.int32, (E, E), 1)
        tri8 = (r8 < c8).astype(jnp.float32)
        off = jnp.dot(capf, tri8, preferred_element_type=jnp.float32)
        off_ref[...] = off
        bvec = lax.broadcasted_iota(jnp.int32, (64, 1), 0) * TB
        ge = (bvec >= off.astype(jnp.int32)).astype(jnp.int32)
        be = jnp.clip(jnp.sum(ge, axis=1, keepdims=True) - 1, 0, E - 1)
        be_ref[...] = be
        ec = lcnt_ref[...] / (N * K) + 1e-08
        loss_ref[...] = (-jnp.sum(ec * jnp.log(ec))).reshape(1, 1)

    @pl.when(t >= NT)
    def _phase_b():
        @pl.when(t == NT)
        def _():
            run_ref[...] = jnp.zeros_like(run_ref)

        d = jnp.exp(m1 - m0)
        g0 = 1.0 / (1.0 + d)
        g1 = d / (1.0 + d)
        rank = jnp.dot(tri_ref[...], a_mat, preferred_element_type=jnp.float32)
        basec = rank + run_ref[...] + off_ref[...]
        pos0_ref[...] = jnp.sum(basec * oh0, axis=1,
                                keepdims=True).astype(jnp.int32)
        pos1_ref[...] = jnp.sum(basec * oh1, axis=1,
                                keepdims=True).astype(jnp.int32)
        run_ref[...] += colsum
        g0x_ref[...] = jnp.broadcast_to(g0, g0x_ref.shape)
        g1x_ref[...] = jnp.broadcast_to(g1, g1x_ref.shape)

        xb = xf.astype(jnp.bfloat16)
        ys = None
        for n in range(sW1_ref.shape[0]):
            h = jnp.dot(xb, sW1_ref[n], preferred_element_type=jnp.float32)
            h = 0.5 * h * (1.0 + lax.erf(h * 0.7071067811865476))
            y = jnp.dot(h.astype(jnp.bfloat16), sW2_ref[n],
                        preferred_element_type=jnp.float32)
            ys = y if ys is None else ys + y
        ys_ref[...] = ys


def _ffn_body(be_sref, buf_ref, W1_ref, W2_ref, gb_ref, y_ref, *, TB):
    b = pl.program_id(0)
    e = be_sref[b]
    xb = buf_ref[...].astype(jnp.bfloat16)
    w1 = W1_ref[pl.ds(e, 1)][0]
    h = jnp.dot(xb, w1, preferred_element_type=jnp.float32)
    h = 0.5 * h * (1.0 + lax.erf(h * 0.7071067811865476))
    w2 = W2_ref[pl.ds(e, 1)][0]
    y = jnp.dot(h.astype(jnp.bfloat16), w2, preferred_element_type=jnp.float32)
    y_ref[...] = y * gb_ref[:, 0:1]


def kernel(x, router_W, router_b, eW1, eb1, eW2, eb2, sW1, sb1, sW2, sb2):
    B, S, DIM = x.shape
    E, _, F = eW1.shape
    NS = sW1.shape[0]
    K = 2
    N = B * S
    T = 512
    NT = N // T
    TB = 256
    CAP = N * K + E * TB
    NB = CAP // TB
    CHUNK = N // _NW

    xf = x.reshape(N, DIM)
    rb = router_b.reshape(1, E)
    tri = (jnp.arange(T)[:, None] > jnp.arange(T)[None, :]
           ).astype(jnp.float32)
    sW1b = sW1.astype(jnp.bfloat16)
    sW2b = sW2.astype(jnp.bfloat16)
    W1e = eW1.astype(jnp.bfloat16)
    W2e = eW2.astype(jnp.bfloat16)

    meta = functools.partial(_meta_body, T=T, E=E, NT=NT, N=N, K=K, TB=TB,
                             NB=NB)

    def _blk(t):
        return (lax.select(t < NT, 0, t - NT), 0)

    ys, pos0, pos1, g0x, g1x, be, loss = pl.pallas_call(
        meta,
        grid=(2 * NT,),
        in_specs=[
            pl.BlockSpec((T, DIM), lambda t: (lax.rem(t, NT), 0)),
            pl.BlockSpec((DIM, E), lambda t: (0, 0)),
            pl.BlockSpec((1, E), lambda t: (0, 0)),
            pl.BlockSpec((T, T), lambda t: (0, 0)),
            pl.BlockSpec((NS, DIM, F), lambda t: (0, 0, 0)),
            pl.BlockSpec((NS, F, DIM), lambda t: (0, 0, 0)),
        ],
        out_specs=[
            pl.BlockSpec((T, DIM), _blk),
            pl.BlockSpec((T, 1), _blk),
            pl.BlockSpec((T, 1), _blk),
            pl.BlockSpec((T, 128), _blk),
            pl.BlockSpec((T, 128), _blk),
            pl.BlockSpec((64, 1), lambda t: (0, 0)),
            pl.BlockSpec((1, 1), lambda t: (0, 0)),
        ],
        out_shape=[
            jax.ShapeDtypeStruct((N, DIM), jnp.float32),
            jax.ShapeDtypeStruct((N, 1), jnp.int32),
            jax.ShapeDtypeStruct((N, 1), jnp.int32),
            jax.ShapeDtypeStruct((N, 128), jnp.float32),
            jax.ShapeDtypeStruct((N, 128), jnp.float32),
            jax.ShapeDtypeStruct((64, 1), jnp.int32),
            jax.ShapeDtypeStruct((1, 1), jnp.float32),
        ],
        scratch_shapes=[
            pltpu.VMEM((1, E), jnp.float32),
            pltpu.VMEM((1, E), jnp.float32),
            pltpu.VMEM((1, E), jnp.float32),
            pltpu.VMEM((1, E), jnp.float32),
        ],
    )(xf, router_W, rb, tri, sW1b, sW2b)

    p0f = pos0.reshape(N)
    p1f = pos1.reshape(N)
    QS = CHUNK // 4
    pos_sc = jnp.concatenate(
        [p0f.reshape(_NW, 4, QS), p1f.reshape(_NW, 4, QS)],
        axis=1)

    mesh = plsc.VectorSubcoreMesh(core_axis_name="c", subcore_axis_name="s")

    @functools.partial(
        pl.kernel, mesh=mesh,
        out_type=[
            jax.ShapeDtypeStruct((CAP, DIM), jnp.float32),
            jax.ShapeDtypeStruct((CAP, 128), jnp.float32),
        ],
        scratch_types=[
            pltpu.VMEM((2, CHUNK // 4, DIM), jnp.float32),
            pltpu.VMEM((2, CHUNK // 4, 128), jnp.float32),
            pltpu.VMEM((2, CHUNK // 4, 128), jnp.float32),
            pltpu.VMEM((8, CHUNK // 4), jnp.int32),
            pltpu.SemaphoreType.DMA,
            pltpu.SemaphoreType.DMA,
            pltpu.SemaphoreType.DMA,
            pltpu.SemaphoreType.DMA,
        ],
    )
    def _dispatch(x_hbm, psc_hbm, g0_hbm, g1_hbm, buf_hbm, gb_hbm,
                  rows_v, g0_v, g1_v, idx_v, sem0, sem1, sem2, sem3):
        wid = lax.axis_index("s") * _NC + lax.axis_index("c")
        base = wid * CHUNK
        qs = CHUNK // 4
        pltpu.sync_copy(psc_hbm.at[wid], idx_v)
        hs = [None] * 4
        for q in range(4):
            b = q % 2
            if q >= 2:
                for h in hs[q - 2]:
                    h.wait()
            pltpu.sync_copy(x_hbm.at[pl.ds(base + q * qs, qs)], rows_v.at[b])
            pltpu.sync_copy(g0_hbm.at[pl.ds(base + q * qs, qs)], g0_v.at[b])
            pltpu.sync_copy(g1_hbm.at[pl.ds(base + q * qs, qs)], g1_v.at[b])
            hs[q] = (
                pltpu.async_copy(rows_v.at[b], buf_hbm.at[idx_v.at[q]], sem0),
                pltpu.async_copy(rows_v.at[b], buf_hbm.at[idx_v.at[4 + q]],
                                 sem1),
                pltpu.async_copy(g0_v.at[b], gb_hbm.at[idx_v.at[q]], sem2),
                pltpu.async_copy(g1_v.at[b], gb_hbm.at[idx_v.at[4 + q]],
                                 sem3),
            )
        for q in (2, 3):
            for h in hs[q]:
                h.wait()

    bufX, gbuf = _dispatch(xf, pos_sc, g0x, g1x)

    ffn = functools.partial(_ffn_body, TB=TB)
    ybuf = pl.pallas_call(
        ffn,
        grid_spec=pltpu.PrefetchScalarGridSpec(
            num_scalar_prefetch=1,
            grid=(NB,),
            in_specs=[
                pl.BlockSpec((TB, DIM), lambda b, be_s: (b, 0)),
                pl.BlockSpec((E, DIM, F), lambda b, be_s: (0, 0, 0)),
                pl.BlockSpec((E, F, DIM), lambda b, be_s: (0, 0, 0)),
                pl.BlockSpec((TB, 128), lambda b, be_s: (b, 0)),
            ],
            out_specs=pl.BlockSpec((TB, DIM), lambda b, be_s: (b, 0)),
        ),
        out_shape=jax.ShapeDtypeStruct((CAP, DIM), jnp.float32),
    )(be.reshape(64), bufX, W1e, W2e, gbuf)

    @functools.partial(
        pl.kernel, mesh=mesh,
        out_type=jax.ShapeDtypeStruct((N, DIM), jnp.float32),
        scratch_types=[
            pltpu.VMEM((CHUNK,), jnp.int32),
            pltpu.VMEM((CHUNK,), jnp.int32),
            pltpu.VMEM((2, 16, DIM), jnp.float32),
            pltpu.VMEM((2, 16, DIM), jnp.float32),
            pltpu.VMEM((2, 16, DIM), jnp.float32),
            pltpu.SemaphoreType.DMA,
            pltpu.SemaphoreType.DMA,
            pltpu.SemaphoreType.DMA,
        ],
    )
    def _combine(ybuf_hbm, ys_hbm, p0_hbm, p1_hbm, out_hbm,
                 p0v, p1v, r0, r1, ov, sem0, sem1, semw):
        wid = lax.axis_index("s") * _NC + lax.axis_index("c")
        base = wid * CHUNK
        pltpu.sync_copy(p0_hbm.at[pl.ds(base, CHUNK)], p0v)
        pltpu.sync_copy(p1_hbm.at[pl.ds(base, CHUNK)], p1v)
        nch = CHUNK // 16
        cps = [None] * nch
        wrs = [None] * nch

        def _start(c):
            p = c % 2
            cp0 = pltpu.async_copy(
                ybuf_hbm.at[p0v.at[pl.ds(c * 16, 16)]], r0.at[p], sem0)
            cp1 = pltpu.async_copy(
                ybuf_hbm.at[p1v.at[pl.ds(c * 16, 16)]], r1.at[p], sem1)
            return (cp0, cp1)

        cps[0] = _start(0)
        for c in range(nch):
            p = c % 2
            if c + 1 < nch:
                if c >= 1:
                    wrs[c - 1].wait()
                cps[c + 1] = _start(c + 1)
            pltpu.sync_copy(ys_hbm.at[pl.ds(base + c * 16, 16)], ov.at[p])
            cps[c][0].wait()
            cps[c][1].wait()

            def _add(j, _):
                row = j // (DIM // 16)
                col = lax.rem(j, DIM // 16) * 16
                ov[p, row, pl.ds(col, 16)] = (ov[p, row, pl.ds(col, 16)]
                                              + r0[p, row, pl.ds(col, 16)]
                                              + r1[p, row, pl.ds(col, 16)])
                return 0

            lax.fori_loop(0, 16 * (DIM // 16), _add, 0, unroll=8)
            wrs[c] = pltpu.async_copy(
                ov.at[p], out_hbm.at[pl.ds(base + c * 16, 16)], semw)
        wrs[nch - 2].wait()
        wrs[nch - 1].wait()

    out = _combine(ybuf, ys, p0f, p1f)
    return out.reshape(B, S, DIM), loss[0, 0]
